# baseline (speedup 1.0000x reference)
import functools

import jax
import jax.numpy as jnp
from jax import lax
from jax.experimental import pallas as pl
from jax.experimental.pallas import tpu as pltpu
from jax.experimental.pallas import tpu_sc as plsc

BATCH = 16384
EMB = 32
_NC = 2
_NS = 16
_NW = _NC * _NS
_BPW = BATCH // _NW
_CHUNK = 128
_NCHUNK = _BPW // _CHUNK


def _sc_gather_body(item_tab, user_tab, dur_tab, ids_hbm,
                    item_out, user_out, dur_out,
                    idx_v, rows_i, rows_u, rows_d, sem):
  wid = lax.axis_index("s") * _NC + lax.axis_index("c")
  base = wid * _BPW
  row0 = wid * _NCHUNK
  pltpu.sync_copy(ids_hbm.at[:, pl.ds(row0, _NCHUNK), :], idx_v)
  copies = []
  for j in range(_NCHUNK):
    sl = pl.ds(j * _CHUNK, _CHUNK)
    copies.append(pltpu.async_copy(item_tab.at[idx_v.at[0, j]], rows_i.at[sl], sem))
    copies.append(pltpu.async_copy(user_tab.at[idx_v.at[1, j]], rows_u.at[sl], sem))
    copies.append(pltpu.async_copy(dur_tab.at[idx_v.at[2, j]], rows_d.at[sl], sem))
  for c in copies:
    c.wait()
  out_sl = pl.ds(base, _BPW)
  pltpu.sync_copy(rows_i, item_out.at[out_sl])
  pltpu.sync_copy(rows_u, user_out.at[out_sl])
  pltpu.sync_copy(rows_d, dur_out.at[out_sl])


def _mlp_body(item_ref, user_ref, dur_ref, w1_ref, b1_ref, w2_ref, b2_ref,
              w3_ref, b3_ref, wo_ref, bo_ref, out_ref):
  f32 = jnp.float32
  h = jnp.dot(item_ref[...], w1_ref[0:EMB, :], preferred_element_type=f32)
  h += jnp.dot(user_ref[...], w1_ref[EMB:2 * EMB, :], preferred_element_type=f32)
  h += jnp.dot(dur_ref[...], w1_ref[2 * EMB:3 * EMB, :], preferred_element_type=f32)
  h = jnp.maximum(h + b1_ref[...], 0.0)
  h = jnp.maximum(jnp.dot(h, w2_ref[...], preferred_element_type=f32) + b2_ref[...], 0.0)
  h = jnp.maximum(jnp.dot(h, w3_ref[...], preferred_element_type=f32) + b3_ref[...], 0.0)
  z = jnp.dot(h, wo_ref[...], preferred_element_type=f32) + bo_ref[...]
  out_ref[...] = 1.0 / (1.0 + jnp.exp(-z))


def kernel(user_id, item_id, duration, is_training, item_table, user_table,
           dur_table, W1, b1, W2, b2, W3, b3, Wo, bo):
  del is_training

  ids = jnp.stack([
      item_id.astype(jnp.int32).reshape(BATCH // _CHUNK, _CHUNK),
      user_id.astype(jnp.int32).reshape(BATCH // _CHUNK, _CHUNK),
      duration.astype(jnp.int32).reshape(BATCH // _CHUNK, _CHUNK),
  ])

  mesh = plsc.VectorSubcoreMesh(core_axis_name="c", subcore_axis_name="s")
  emb_out = jax.ShapeDtypeStruct((BATCH, EMB), jnp.float32)
  gather = functools.partial(
      pl.kernel,
      mesh=mesh,
      compiler_params=pltpu.CompilerParams(use_tc_tiling_on_sc=False),
      out_type=(emb_out, emb_out, emb_out),
      scratch_types=[
          pltpu.VMEM((3, _NCHUNK, _CHUNK), jnp.int32),
          pltpu.VMEM((_BPW, EMB), jnp.float32),
          pltpu.VMEM((_BPW, EMB), jnp.float32),
          pltpu.VMEM((_BPW, EMB), jnp.float32),
          pltpu.SemaphoreType.DMA,
      ],
  )(_sc_gather_body)
  item_emb, user_emb, dur_emb = gather(item_table, user_table, dur_table, ids)

  bm = 2048
  grid = (BATCH // bm,)
  full = lambda shape: pl.BlockSpec(shape, lambda i: (0,) * len(shape))
  out = pl.pallas_call(
      _mlp_body,
      grid=grid,
      in_specs=[
          pl.BlockSpec((bm, EMB), lambda i: (i, 0)),
          pl.BlockSpec((bm, EMB), lambda i: (i, 0)),
          pl.BlockSpec((bm, EMB), lambda i: (i, 0)),
          full((3 * EMB, 128)),
          full((1, 128)),
          full((128, 64)),
          full((1, 64)),
          full((64, 32)),
          full((1, 32)),
          full((32, 2)),
          full((1, 2)),
      ],
      out_specs=pl.BlockSpec((bm, 2), lambda i: (i, 0)),
      out_shape=jax.ShapeDtypeStruct((BATCH, 2), jnp.float32),
  )(item_emb, user_emb, dur_emb, W1, b1.reshape(1, 128), W2,
    b2.reshape(1, 64), W3, b3.reshape(1, 32), Wo, bo.reshape(1, 2))
  return out
